# baseline (speedup 1.0000x reference)
import functools

import jax
import jax.numpy as jnp
from jax import lax
from jax.experimental import pallas as pl
from jax.experimental.pallas import tpu as pltpu
from jax.experimental.pallas import tpu_sc as plsc

VOCAB1 = 100001
D = 64
DP = 128
B, L = 4096, 50
NC, NS = 2, 16
NW = NC * NS
B_PER_W = B // NW
CHB = 4
NCH = B_PER_W // CHB
NV = D // 16


def _gather_grid(table_hbm, idx_hbm, out_hbm, idx_v, rows_v, comp_v,
                 g0, g1, w0):
    wid = lax.axis_index("s") * NC + lax.axis_index("c")
    bbase = wid * B_PER_W
    gsem = (g0, g1)

    pltpu.sync_copy(idx_hbm.at[pl.ds(bbase, B_PER_W)], idx_v)

    def start_gathers(j, b):
        return [
            pltpu.async_copy(
                table_hbm.at[idx_v.at[j * CHB + k]],
                rows_v.at[b].at[k],
                gsem[b],
            )
            for k in range(CHB)
        ]

    def compact(b):
        def body(m, carry):
            k = m // L
            l = m % L
            for c in range(NV):
                comp_v[k, l, pl.ds(c * 16, 16)] = (
                    rows_v[b, k, l, pl.ds(c * 16, 16)]
                )
            return carry
        lax.fori_loop(0, CHB * L, body, 0)

    def start_writeback(j):
        return pltpu.async_copy(
            comp_v,
            out_hbm.at[pl.ds(bbase + j * CHB, CHB)],
            w0,
        )

    gh = [None] * NCH
    wh = [None] * NCH
    gh[0] = start_gathers(0, 0)
    for j in range(NCH):
        b = j % 2
        for h in gh[j]:
            h.wait()
        if j + 1 < NCH:
            gh[j + 1] = start_gathers(j + 1, 1 - b)
        if j >= 1:
            wh[j - 1].wait()
        compact(b)
        wh[j] = start_writeback(j)
    wh[NCH - 1].wait()


def kernel(indices, vectors):
    table = jnp.pad(vectors, ((0, 0), (0, DP - D)))
    idx = indices.astype(jnp.int32)
    mesh = plsc.VectorSubcoreMesh(core_axis_name="c", subcore_axis_name="s")
    run = functools.partial(
        pl.kernel,
        mesh=mesh,
        out_type=jax.ShapeDtypeStruct((B, L, D), jnp.float32),
        scratch_types=[
            pltpu.VMEM((B_PER_W, L), jnp.int32),
            pltpu.VMEM((2, CHB, L, DP), jnp.float32),
            pltpu.VMEM((CHB, L, D), jnp.float32),
            pltpu.SemaphoreType.DMA,
            pltpu.SemaphoreType.DMA,
            pltpu.SemaphoreType.DMA,
        ],
    )(_gather_grid)
    return run(table, idx)


# speedup vs baseline: 1.0106x; 1.0106x over previous
import functools

import jax
import jax.numpy as jnp
from jax import lax
from jax.experimental import pallas as pl
from jax.experimental.pallas import tpu as pltpu
from jax.experimental.pallas import tpu_sc as plsc

VOCAB1 = 100001
D = 64
DP = 128
B, L = 4096, 50
NC, NS = 2, 16
NW = NC * NS
B_PER_W = B // NW
CHB = 4
NCH = B_PER_W // CHB
NV = D // 16


def _gather_grid(table_hbm, idx_hbm, out_hbm, idx_v, rows_v, comp_v,
                 g0, g1, w0):
    wid = lax.axis_index("s") * NC + lax.axis_index("c")
    bbase = wid * B_PER_W
    gsem = (g0, g1)

    pltpu.sync_copy(idx_hbm.at[pl.ds(bbase, B_PER_W)], idx_v)

    def start_gathers(j, b):
        return [
            pltpu.async_copy(
                table_hbm.at[idx_v.at[j * CHB + k]],
                rows_v.at[b].at[k],
                gsem[b],
            )
            for k in range(CHB)
        ]

    def compact(b):
        def body(l, carry):
            for k in range(CHB):
                for c in range(NV):
                    comp_v[k, l, pl.ds(c * 16, 16)] = (
                        rows_v[b, k, l, pl.ds(c * 16, 16)]
                    )
            return carry
        lax.fori_loop(0, L, body, 0, unroll=2)

    def start_writeback(j):
        return pltpu.async_copy(
            comp_v,
            out_hbm.at[pl.ds(bbase + j * CHB, CHB)],
            w0,
        )

    gh = [None] * NCH
    wh = [None] * NCH
    gh[0] = start_gathers(0, 0)
    for j in range(NCH):
        b = j % 2
        for h in gh[j]:
            h.wait()
        if j + 1 < NCH:
            gh[j + 1] = start_gathers(j + 1, 1 - b)
        if j >= 1:
            wh[j - 1].wait()
        compact(b)
        wh[j] = start_writeback(j)
    wh[NCH - 1].wait()


def kernel(indices, vectors):
    table = jnp.pad(vectors, ((0, 0), (0, DP - D)))
    idx = indices.astype(jnp.int32)
    mesh = plsc.VectorSubcoreMesh(core_axis_name="c", subcore_axis_name="s")
    run = functools.partial(
        pl.kernel,
        mesh=mesh,
        out_type=jax.ShapeDtypeStruct((B, L, D), jnp.float32),
        scratch_types=[
            pltpu.VMEM((B_PER_W, L), jnp.int32),
            pltpu.VMEM((2, CHB, L, DP), jnp.float32),
            pltpu.VMEM((CHB, L, D), jnp.float32),
            pltpu.SemaphoreType.DMA,
            pltpu.SemaphoreType.DMA,
            pltpu.SemaphoreType.DMA,
        ],
    )(_gather_grid)
    return run(table, idx)
